# baseline (speedup 1.0000x reference)
import jax
import jax.numpy as jnp
from jax import lax
from jax.experimental import pallas as pl
from jax.experimental.pallas import tpu as pltpu
from jax.experimental.pallas import tpu_sc as plsc

R = 32
N = 1_000_000
L = 16
SPAN = 124_928
CH = 7_680
NCH = 16
TPC = CH // 128
MINI_OFF = NCH * CH
MINI = SPAN - MINI_OFF
TAIL_OFF = 8 * SPAN
LAST_T = (N // 128) * 128
TAILZ = LAST_T - TAIL_OFF
LASTW = N - LAST_T
ZB = 8_192
NZW = SPAN // ZB
ZREM = SPAN - NZW * ZB
ZFILL = ZB // 16
SENT = 2 ** 30
NEG_INF = float("-inf")


def _hardmax_body(logits, last_in, out, out_last,
                  buf_a, buf_b, loc_v, loc_i, svals_l, sidx_l,
                  zlast_l, olast,
                  szeros, svals_sh, sidx_sh, sidx2_sh,
                  sem_a, sem_b, sem_w, sem_t):
    c = lax.axis_index("c")
    s = lax.axis_index("s")
    g = c * 2 + jnp.right_shift(s, 3)
    h = jnp.bitwise_and(s, 7)
    row0 = g * 8
    span_off = h * SPAN
    is_merger = h == 0
    iota16 = lax.iota(jnp.int32, L)
    zv = jnp.zeros((L,), jnp.float32)

    gdn = lax.GatherDimensionNumbers(
        offset_dims=(), collapsed_slice_dims=(0,), start_index_map=(0,))

    def shuffle(x, perm):
        return lax.gather(
            x, perm[:, None], gdn, slice_sizes=(1,),
            mode=lax.GatherScatterMode.PROMISE_IN_BOUNDS)

    def bfly(x, op):
        for k in (8, 4, 2, 1):
            x = op(x, shuffle(x, jnp.bitwise_xor(iota16, k)))
        return x

    def read_chunk(ck, buf, sem):
        pltpu.make_async_copy(
            logits.at[pl.ds(row0, 8), pl.ds(span_off + ck * CH, CH)],
            buf, sem).start()

    def wait_chunk(buf, sem):
        pltpu.make_async_copy(
            logits.at[pl.ds(row0, 8), pl.ds(span_off, CH)], buf, sem).wait()

    read_chunk(0, buf_a, sem_a)

    def zfill(i, _):
        for q in range(8):
            buf_b[q, pl.ds(i * L, L)] = zv
        return 0

    lax.fori_loop(0, ZFILL // L, zfill, 0)
    pltpu.sync_copy(buf_b.at[:, pl.ds(0, ZFILL)],
                    szeros.at[:, pl.ds(s * ZFILL, ZFILL)])
    plsc.subcore_barrier()

    def start_zw(w):
        pltpu.make_async_copy(
            szeros, out.at[pl.ds(row0, 8), pl.ds(span_off + w * ZB, ZB)],
            sem_w).start()

    start_zw(0)
    start_zw(1)

    @pl.when(is_merger)
    def _():
        pltpu.make_async_copy(
            szeros.at[:, pl.ds(0, TAILZ)],
            out.at[pl.ds(row0, 8), pl.ds(TAIL_OFF, TAILZ)],
            sem_w).start()

    pltpu.make_async_copy(
        last_in.at[pl.ds(row0, 8), :], zlast_l, sem_t).start()

    read_chunk(1, buf_b, sem_b)

    def scan_block(block_off, tiles, buf, carry):
        def tile(j, cr):
            col0 = block_off + j * 128
            new = []
            for q in range(8):
                cur, base = cr[2 * q], cr[2 * q + 1]
                m = buf[q, pl.ds(j * 128, L)]
                for v in range(1, 8):
                    m = jnp.maximum(m, buf[q, pl.ds(j * 128 + v * L, L)])
                upd = m > cur
                new.append(jnp.maximum(cur, m))
                new.append(jnp.where(upd, col0, base))
            return tuple(new)

        return lax.fori_loop(0, tiles, tile, carry)

    init = []
    for _ in range(8):
        init.append(jnp.full((L,), NEG_INF, jnp.float32))
        init.append(jnp.full((L,), jnp.int32(0), jnp.int32))

    def mini_copy():
        return pltpu.make_async_copy(
            logits.at[pl.ds(row0, 8), pl.ds(span_off + MINI_OFF, MINI)],
            buf_a.at[:, pl.ds(0, MINI)], sem_a)

    def pair(k, carry):
        wait_chunk(buf_a, sem_a)
        carry = scan_block(span_off + 2 * k * CH, TPC, buf_a, carry)

        @pl.when(2 * k + 2 < NCH)
        def _():
            read_chunk(2 * k + 2, buf_a, sem_a)

        @pl.when(k == NCH // 2 - 1)
        def _():
            mini_copy().start()

        @pl.when(2 * k + 2 < NZW)
        def _():
            start_zw(2 * k + 2)

        wait_chunk(buf_b, sem_b)
        carry = scan_block(span_off + (2 * k + 1) * CH, TPC, buf_b, carry)

        @pl.when(2 * k + 3 < NCH)
        def _():
            read_chunk(2 * k + 3, buf_b, sem_b)

        @pl.when(2 * k + 3 < NZW)
        def _():
            start_zw(2 * k + 3)

        @pl.when(k == NCH // 2 - 1)
        def _():
            pltpu.make_async_copy(
                szeros.at[:, pl.ds(0, ZREM)],
                out.at[pl.ds(row0, 8),
                       pl.ds(span_off + NZW * ZB, ZREM)],
                sem_w).start()

        return carry

    carry = lax.fori_loop(0, NCH // 2, pair, tuple(init))

    mini_copy().wait()
    carry = scan_block(span_off + MINI_OFF, MINI // 128, buf_a, carry)

    pack_v = jnp.full((L,), NEG_INF, jnp.float32)
    pack_i = jnp.full((L,), jnp.int32(0), jnp.int32)
    for q in range(8):
        cur, base = carry[2 * q], carry[2 * q + 1]
        g16 = bfly(cur, jnp.maximum)
        t16 = bfly(jnp.where(cur == g16, base, SENT), jnp.minimum)
        pack_v = jnp.where(iota16 == q, g16, pack_v)
        pack_i = jnp.where(iota16 == q, t16, pack_i)
    loc_v[...] = pack_v
    loc_i[...] = pack_i

    pltpu.sync_copy(loc_v, svals_sh.at[pl.ds(s * L, L)])
    pltpu.sync_copy(loc_i, sidx_sh.at[pl.ds(s * L, L)])

    pltpu.make_async_copy(
        logits.at[pl.ds(row0, 8), pl.ds(TAIL_OFF, TAILZ)],
        buf_b.at[:, pl.ds(0, TAILZ)], sem_t).start()

    plsc.subcore_barrier()

    s0 = jnp.bitwise_and(s, 8)
    pltpu.sync_copy(svals_sh.at[pl.ds(s0 * L, 8 * L)], svals_l)
    pltpu.sync_copy(sidx_sh.at[pl.ds(s0 * L, 8 * L)], sidx_l)
    pltpu.make_async_copy(
        last_in.at[pl.ds(row0, 8), :], zlast_l, sem_t).wait()
    pltpu.make_async_copy(
        logits.at[pl.ds(row0, 8), pl.ds(TAIL_OFF, TAILZ)],
        buf_b.at[:, pl.ds(0, TAILZ)], sem_t).wait()

    tail_v = jnp.full((L,), NEG_INF, jnp.float32)
    tail_i = jnp.full((L,), jnp.int32(0), jnp.int32)
    for q in range(8):
        tcur = jnp.full((L,), NEG_INF, jnp.float32)
        tidx = jnp.full((L,), SENT, jnp.int32)
        for v in range(TAILZ // L):
            val = buf_b[q, pl.ds(v * L, L)]
            upd = val > tcur
            tcur = jnp.maximum(tcur, val)
            tidx = jnp.where(upd, TAIL_OFF + v * L + iota16, tidx)
        for v in range(LASTW // L):
            val = zlast_l[q, pl.ds(v * L, L)]
            upd = val > tcur
            tcur = jnp.maximum(tcur, val)
            tidx = jnp.where(upd, LAST_T + v * L + iota16, tidx)
        tg16 = bfly(tcur, jnp.maximum)
        tp16 = bfly(jnp.where(tcur == tg16, tidx, SENT), jnp.minimum)
        tail_v = jnp.where(iota16 == q, tg16, tail_v)
        tail_i = jnp.where(iota16 == q, tp16, tail_i)

    best_v = svals_l[pl.ds(0, L)]
    best_t = sidx_l[pl.ds(0, L)]
    for hh in range(1, 8):
        vv = svals_l[pl.ds(hh * L, L)]
        ii = sidx_l[pl.ds(hh * L, L)]
        upd = vv > best_v
        best_v = jnp.maximum(best_v, vv)
        best_t = jnp.where(upd, ii, best_t)
    upd = tail_v > best_v
    best_t = jnp.where(upd, tail_i, best_t)

    tb_h = bfly(jnp.where(iota16 == h, best_t, SENT), jnp.minimum)[0]
    tbq_h = pl.multiple_of(jnp.where(tb_h < TAIL_OFF, tb_h, 0), 128)
    refetch = pltpu.make_async_copy(
        logits.at[pl.ds(row0, 8), pl.ds(tbq_h, 128)],
        buf_b.at[:, pl.ds(1024, 128)], sem_t)
    refetch.start()
    refetch.wait()

    ex16 = jnp.full((L,), jnp.int32(0), jnp.int32)
    for q in range(8):
        tb = best_t[q]
        tbq = jnp.where(tb < TAIL_OFF, tb, 0)
        bq16 = shuffle(best_v, jnp.full((L,), q, jnp.int32))
        scan = jnp.full((L,), SENT, jnp.int32)
        for v in range(8):
            val = buf_b[q, pl.ds(1024 + v * L, L)]
            scan = jnp.minimum(
                scan, jnp.where(val == bq16, tbq + v * L + iota16, SENT))
        sc16 = bfly(scan, jnp.minimum)
        ex = jnp.where(tb < TAIL_OFF, sc16[0], tb)
        ex16 = jnp.where(iota16 == q, ex, ex16)

    loc_i[...] = jnp.where(iota16 == h, ex16, 0)
    pltpu.sync_copy(loc_i, sidx2_sh.at[pl.ds(s * L, L)])
    plsc.subcore_barrier()

    pltpu.sync_copy(sidx2_sh.at[pl.ds(s0 * L, 8 * L)], sidx_l)
    exact = jnp.full((L,), jnp.int32(0), jnp.int32)
    for q in range(8):
        vq = sidx_l[pl.ds(q * L, L)]
        exact = jnp.where(iota16 == q, vq, exact)

    ph = bfly(jnp.where(iota16 == h, exact, SENT), jnp.minimum)[0]
    th = jnp.right_shift(ph, 7)
    for qq in range(8):
        pqq = exact[qq]
        cin = jnp.where(jnp.right_shift(pqq, 7) == th,
                        jnp.bitwise_and(pqq, 127), -1)
        for v in range(8):
            buf_a[qq, pl.ds(v * L, L)] = jnp.where(
                v * L + iota16 == cin,
                jnp.float32(1.0), jnp.float32(0.0))
    col0 = pl.multiple_of(th * 128, 128)

    for _ in range(NZW):
        pltpu.make_async_copy(
            szeros, out.at[pl.ds(row0, 8), pl.ds(0, ZB)], sem_w).wait()
    pltpu.make_async_copy(
        szeros.at[:, pl.ds(0, ZREM)],
        out.at[pl.ds(row0, 8), pl.ds(0, ZREM)], sem_w).wait()

    @pl.when(is_merger)
    def _():
        pltpu.make_async_copy(
            szeros.at[:, pl.ds(0, TAILZ)],
            out.at[pl.ds(row0, 8), pl.ds(0, TAILZ)], sem_w).wait()

    plsc.subcore_barrier()

    @pl.when(col0 < LAST_T)
    def _():
        pltpu.make_async_copy(
            buf_a.at[:, pl.ds(0, 128)],
            out.at[pl.ds(row0, 8), pl.ds(col0, 128)], sem_w).start()

    @pl.when(is_merger)
    def _():
        for qq in range(8):
            pqq = exact[qq]
            cin = jnp.where(pqq >= LAST_T, pqq - LAST_T, -1)
            for v in range(LASTW // L):
                olast[qq, pl.ds(v * L, L)] = jnp.where(
                    v * L + iota16 == cin,
                    jnp.float32(1.0), jnp.float32(0.0))
        pltpu.sync_copy(olast, out_last.at[pl.ds(row0, 8), :])

    @pl.when(col0 < LAST_T)
    def _():
        pltpu.make_async_copy(
            buf_a.at[:, pl.ds(0, 128)],
            out.at[pl.ds(row0, 8), pl.ds(0, 128)], sem_w).wait()


def kernel(logits):
    mesh = plsc.VectorSubcoreMesh(
        core_axis_name="c", subcore_axis_name="s",
        num_cores=2, num_subcores=16)
    run = pl.kernel(
        _hardmax_body,
        out_type=(
            jax.ShapeDtypeStruct((R, N), jnp.float32),
            jax.ShapeDtypeStruct((R, LASTW), jnp.float32),
        ),
        mesh=mesh,
        scratch_types=[
            pltpu.VMEM((8, CH), jnp.float32),
            pltpu.VMEM((8, CH), jnp.float32),
            pltpu.VMEM((L,), jnp.float32),
            pltpu.VMEM((L,), jnp.int32),
            pltpu.VMEM((8 * L,), jnp.float32),
            pltpu.VMEM((8 * L,), jnp.int32),
            pltpu.VMEM((8, LASTW), jnp.float32),
            pltpu.VMEM((8, LASTW), jnp.float32),
            pltpu.VMEM_SHARED((8, ZB), jnp.float32),
            pltpu.VMEM_SHARED((16 * L,), jnp.float32),
            pltpu.VMEM_SHARED((16 * L,), jnp.int32),
            pltpu.VMEM_SHARED((16 * L,), jnp.int32),
            pltpu.SemaphoreType.DMA,
            pltpu.SemaphoreType.DMA,
            pltpu.SemaphoreType.DMA,
            pltpu.SemaphoreType.DMA,
        ],
    )
    last_in = lax.slice(logits, (0, LAST_T), (R, N))
    out, out_last = run(logits, last_in)
    return lax.dynamic_update_slice(out, out_last, (0, LAST_T))
